# baseline (speedup 1.0000x reference)
import functools

import jax
import jax.numpy as jnp
from jax import lax
from jax.experimental import pallas as pl
from jax.experimental.pallas import tpu as pltpu

_EPS = 1e-5
_TB = 2


def _ln_row_kernel(x_ref, o_ref, *, inv_n):
    tb, s, h, w = x_ref.shape
    x = x_ref[...].reshape(tb, s * h, w)
    s1 = jnp.sum(x, axis=1, keepdims=True)
    q1 = jnp.sum(x * x, axis=1, keepdims=True)
    sm = jnp.sum(s1, axis=2, keepdims=True)
    qm = jnp.sum(q1, axis=2, keepdims=True)
    mean = sm * inv_n
    var = jnp.maximum(qm * inv_n - mean * mean, 0.0)
    scale = lax.rsqrt(var + _EPS)
    shift = -mean * scale
    o_ref[...] = (x * scale + shift).reshape(tb, s, h, w)


def kernel(x):
    b, s, h, w = (int(d) for d in x.shape)
    n = s * h * w

    tb = _TB
    while b % tb:
        tb //= 2

    return pl.pallas_call(
        functools.partial(_ln_row_kernel, inv_n=1.0 / float(n)),
        out_shape=jax.ShapeDtypeStruct((b, s, h, w), x.dtype),
        grid=(b // tb,),
        in_specs=[pl.BlockSpec((tb, s, h, w), lambda i: (i, 0, 0, 0))],
        out_specs=pl.BlockSpec((tb, s, h, w), lambda i: (i, 0, 0, 0)),
        compiler_params=pltpu.CompilerParams(
            dimension_semantics=("parallel",),
            vmem_limit_bytes=50 * 1024 * 1024,
        ),
        cost_estimate=pl.CostEstimate(
            flops=7 * b * n,
            transcendentals=b,
            bytes_accessed=2 * b * n * 4,
        ),
    )(x)


# speedup vs baseline: 13.2410x; 13.2410x over previous
import functools

import jax
import jax.numpy as jnp
from jax import lax
from jax.experimental import pallas as pl
from jax.experimental.pallas import tpu as pltpu

_EPS = 1e-5
_TB = 8


def _ln_row_kernel(x_ref, o_ref, *, inv_n):
    tb, h, w, c = x_ref.shape
    x = x_ref[...].reshape(tb, h * w, c)
    s1 = jnp.sum(x, axis=1, keepdims=True)
    q1 = jnp.sum(x * x, axis=1, keepdims=True)
    sm = jnp.sum(s1, axis=2, keepdims=True)
    qm = jnp.sum(q1, axis=2, keepdims=True)
    mean = sm * inv_n
    var = jnp.maximum(qm * inv_n - mean * mean, 0.0)
    scale = lax.rsqrt(var + _EPS)
    shift = -mean * scale
    o_ref[...] = (x * scale + shift).reshape(tb, h, w, c)


def kernel(x):
    b, c, h, w = (int(d) for d in x.shape)
    n = c * h * w

    tb = _TB
    while b % tb:
        tb //= 2

    y = jnp.transpose(x, (0, 2, 3, 1))
    out = pl.pallas_call(
        functools.partial(_ln_row_kernel, inv_n=1.0 / float(n)),
        out_shape=jax.ShapeDtypeStruct((b, h, w, c), x.dtype),
        grid=(b // tb,),
        in_specs=[pl.BlockSpec((tb, h, w, c), lambda i: (i, 0, 0, 0))],
        out_specs=pl.BlockSpec((tb, h, w, c), lambda i: (i, 0, 0, 0)),
        compiler_params=pltpu.CompilerParams(
            dimension_semantics=("parallel",),
            vmem_limit_bytes=50 * 1024 * 1024,
        ),
        cost_estimate=pl.CostEstimate(
            flops=7 * b * n,
            transcendentals=b,
            bytes_accessed=2 * b * n * 4,
        ),
    )(y)
    return jnp.transpose(out, (0, 3, 1, 2))
